# baseline (speedup 1.0000x reference)
import dataclasses
import functools

import jax
import jax.numpy as jnp
from jax import lax
from jax.experimental import pallas as pl
from jax.experimental.pallas import tpu as pltpu
from jax.experimental.pallas import tpu_sc as plsc

N = 50000
E = 800000
D = 64
H = 32
NS = 16
EPS = E // NS
CHUNK = 80
SUPE = 2000
NSUP = EPS // SUPE
SUPC = SUPE // CHUNK
RING = 4
MAIN_T = SUPC // RING
TAIL_P = MAIN_T * RING
BLKR = 400
NBLK = N // BLKR
LANES = 16


_RB = 5000


def _prep_body(x_ref, W1, b1, W2, b2, mlp_ref):
    x = x_ref[...]
    h = jnp.maximum(jnp.dot(x, W1[...], preferred_element_type=jnp.float32)
                    + b1[...], 0.0)
    m = jnp.maximum(jnp.dot(h, W2[...], preferred_element_type=jnp.float32)
                    + b2[...], 0.0)
    mlp_ref[0] = m[:, :H]
    mlp_ref[1] = m[:, H:]


def _prep_one(x, W1, b1, W2, b2):
    row_spec = pl.BlockSpec((_RB, D), lambda i: (i, 0))
    w_spec = pl.BlockSpec((D, D), lambda i: (0, 0))
    b_spec = pl.BlockSpec((1, D), lambda i: (0, 0))
    cat_spec = pl.BlockSpec((2, _RB, H), lambda i: (0, i, 0))
    return pl.pallas_call(
        _prep_body,
        grid=(N // _RB,),
        in_specs=[row_spec, w_spec, b_spec, w_spec, b_spec],
        out_specs=cat_spec,
        out_shape=jax.ShapeDtypeStruct((2, N, H), jnp.float32),
    )(x, W1, b1.reshape(1, D), W2, b2.reshape(1, D))


def _sc_dir(src_cat, init_feats, gidx, sidx, vals):
    mesh = plsc.VectorSubcoreMesh(core_axis_name="c", subcore_axis_name="s",
                                  num_cores=2, num_subcores=NS)
    out_sd = jax.ShapeDtypeStruct((2 * N, H), jnp.float32)
    cp = pltpu.CompilerParams()
    fields = pltpu.CompilerParams.__dataclass_fields__
    if "needs_layout_passes" in fields:
        cp = dataclasses.replace(cp, needs_layout_passes=False)
    if "use_tc_tiling_on_sc" in fields:
        cp = dataclasses.replace(cp, use_tc_tiling_on_sc=False)

    ring_scratch = []
    for _ in range(RING):
        ring_scratch += [pltpu.VMEM((CHUNK, H), jnp.float32),
                         pltpu.VMEM((CHUNK,), jnp.int32),
                         pltpu.VMEM((CHUNK, LANES), jnp.float32),
                         pltpu.SemaphoreType.DMA,
                         pltpu.SemaphoreType.DMA,
                         pltpu.SemaphoreType.DMA]

    @functools.partial(
        pl.kernel,
        out_type=out_sd,
        mesh=mesh,
        compiler_params=cp,
        scratch_types=[
            pltpu.VMEM((SUPE,), jnp.int32),
            pltpu.VMEM((SUPE,), jnp.int32),
            pltpu.VMEM_SHARED((N, H), jnp.float32),
        ] + ring_scratch,
    )
    def kern(src_hbm, init_hbm, gidx_hbm, sidx_hbm, vals_hbm, out_hbm,
             gidxB, sidxB, accum, *ring_flat):
        c = lax.axis_index("c")
        s = lax.axis_index("s")
        coff = c * N
        ring = [tuple(ring_flat[6 * r:6 * r + 6]) for r in range(RING)]

        def stage(slot, q, sb):
            pltpu.async_copy(vals_hbm.at[pl.ds(sb + q * CHUNK, CHUNK)],
                             slot[2], slot[5])
            for k in range(CHUNK // LANES):
                sl = pl.ds(q * CHUNK + k * LANES, LANES)
                dl = pl.ds(k * LANES, LANES)
                slot[1][dl] = sidxB[sl]
                gidxB[sl] = gidxB[sl] + coff

        def issue_gather(q, slot):
            pltpu.async_copy(src_hbm.at[gidxB.at[pl.ds(q * CHUNK, CHUNK)]],
                             slot[0], slot[3])

        def wait_gather(slot):
            pltpu.make_async_copy(src_hbm.at[gidxB.at[pl.ds(0, CHUNK)]],
                                  slot[0], slot[3]).wait()
            pltpu.make_async_copy(vals_hbm.at[pl.ds(0, CHUNK)],
                                  slot[2], slot[5]).wait()

        def issue_scatter(slot):
            pltpu.async_copy(slot[0], accum.at[slot[1]], slot[4], add=True)

        def wait_scatter(slot):
            pltpu.make_async_copy(slot[0], accum.at[slot[1]], slot[4]).wait()

        def scale(slot):
            rows_b = slot[0]
            vals_b = slot[2]

            @pl.loop(0, CHUNK // LANES)
            def _(i):
                for k in range(LANES):
                    j = i * LANES + k
                    v = vals_b[j]
                    lo = pl.ds(0, LANES)
                    hi = pl.ds(LANES, LANES)
                    rows_b[j, lo] = rows_b[j, lo] * v
                    rows_b[j, hi] = rows_b[j, hi] * v

        def chunk(p, slot, slot2, prepare, sb):
            wait_gather(slot)
            if prepare:
                q = p + 2

                @pl.when(q < SUPC)
                def _():
                    @pl.when(p >= 2)
                    def _():
                        wait_scatter(slot2)

                    stage(slot2, q, sb)
                    issue_gather(q, slot2)

            scale(slot)
            issue_scatter(slot)

        @pl.loop(0, (NBLK + NS - 1) // NS)
        def _(i):
            b = s + i * NS

            @pl.when(b < NBLK)
            def _():
                pltpu.sync_copy(
                    init_hbm.at[pl.ds(b * BLKR, BLKR), pl.ds(c * H, H)],
                    accum.at[pl.ds(b * BLKR, BLKR)])

        plsc.subcore_barrier()
        base_e = s * EPS

        @pl.loop(0, NSUP)
        def _(u):
            sb = base_e + u * SUPE
            pltpu.sync_copy(gidx_hbm.at[pl.ds(sb, SUPE)], gidxB)
            pltpu.sync_copy(sidx_hbm.at[pl.ds(sb, SUPE)], sidxB)
            for p0 in range(2):
                stage(ring[p0], p0, sb)
                issue_gather(p0, ring[p0])

            @pl.loop(0, MAIN_T)
            def _(t):
                for r in range(RING):
                    p = t * RING + r
                    chunk(p, ring[r], ring[(r + 2) % RING], True, sb)

            chunk(TAIL_P, ring[TAIL_P % RING], None, False, sb)

            for r in range(RING):
                wait_scatter(ring[r])

        plsc.subcore_barrier()

        @pl.loop(0, (NBLK + NS - 1) // NS)
        def _(i):
            b = s + i * NS

            @pl.when(b < NBLK)
            def _():
                pltpu.sync_copy(accum.at[pl.ds(b * BLKR, BLKR)],
                                out_hbm.at[pl.ds(coff + b * BLKR, BLKR)])

        plsc.subcore_barrier()

    return kern(src_cat, init_feats, gidx, sidx, vals)


_EB = 8000


def _vals16_body(v_ref, out_ref):
    out_ref[...] = jnp.broadcast_to(v_ref[...], (_EB, LANES))


def _vals16(vals):
    return pl.pallas_call(
        _vals16_body,
        grid=(E // _EB,),
        in_specs=[pl.BlockSpec((_EB, 1), lambda i: (i, 0))],
        out_specs=pl.BlockSpec((_EB, LANES), lambda i: (i, 0)),
        out_shape=jax.ShapeDtypeStruct((E, LANES), jnp.float32),
    )(vals.reshape(E, 1))


def _post_body(gc_ref, tilde_ref, W, b, g_ref, beta_ref, ln_ref, j_ref):
    x = jnp.concatenate([gc_ref[0], gc_ref[1]], axis=-1)
    mu = jnp.mean(x, axis=-1, keepdims=True)
    xc = x - mu
    var = jnp.mean(xc * xc, axis=-1, keepdims=True)
    ln = xc * lax.rsqrt(var + 1e-5) * g_ref[...] + beta_ref[...]
    ln_ref[...] = ln
    jin = jnp.concatenate([ln, tilde_ref[...]], axis=-1)
    j_ref[...] = jax.nn.sigmoid(
        jnp.dot(jin, W[...], preferred_element_type=jnp.float32) + b[...])


def _post_one(gc_cat, tilde, jump_W, jump_b, ln_g, ln_b):
    cat_spec = pl.BlockSpec((2, _RB, H), lambda i: (0, i, 0))
    row_spec = pl.BlockSpec((_RB, D), lambda i: (i, 0))
    w_spec = pl.BlockSpec((2 * D, D), lambda i: (0, 0))
    b_spec = pl.BlockSpec((1, D), lambda i: (0, 0))
    out_sd = jax.ShapeDtypeStruct((N, D), jnp.float32)
    return pl.pallas_call(
        _post_body,
        grid=(N // _RB,),
        in_specs=[cat_spec, row_spec, w_spec, b_spec, b_spec, b_spec],
        out_specs=[row_spec, row_spec],
        out_shape=[out_sd, out_sd],
    )(gc_cat, tilde, jump_W, jump_b.reshape(1, D),
      ln_g.reshape(1, D), ln_b.reshape(1, D))


def kernel(cons_features, vars_features, tilde_cons_features,
           tilde_vars_features, edge_indices, edge_attrs,
           cons_W1, cons_b1, cons_W2, cons_b2,
           vars_W1, vars_b1, vars_W2, vars_b2,
           jump_cons_W, jump_cons_b, jump_vars_W, jump_vars_b,
           cons_ln_g, cons_ln_b, vars_ln_g, vars_ln_b):
    rows = edge_indices[0].astype(jnp.int32)
    cols = edge_indices[1].astype(jnp.int32)
    vals16 = _vals16(edge_attrs[:, 0])

    mlp_v_cat = _prep_one(vars_features, vars_W1, vars_b1, vars_W2, vars_b2)
    mlp_c_cat = _prep_one(cons_features, cons_W1, cons_b1, cons_W2, cons_b2)

    gc_cons_flat = _sc_dir(mlp_v_cat.reshape(2 * N, H), cons_features,
                           cols, rows, vals16)
    gc_vars_flat = _sc_dir(mlp_c_cat.reshape(2 * N, H), vars_features,
                           rows, cols, vals16)

    ln_cons, jump_cons = _post_one(gc_cons_flat.reshape(2, N, H),
                                   tilde_cons_features, jump_cons_W,
                                   jump_cons_b, cons_ln_g, cons_ln_b)
    ln_vars, jump_vars = _post_one(gc_vars_flat.reshape(2, N, H),
                                   tilde_vars_features, jump_vars_W,
                                   jump_vars_b, vars_ln_g, vars_ln_b)

    return (ln_cons, ln_vars, jump_cons, jump_vars)


# speedup vs baseline: 1.5112x; 1.5112x over previous
import dataclasses
import functools

import jax
import jax.numpy as jnp
from jax import lax
from jax.experimental import pallas as pl
from jax.experimental.pallas import tpu as pltpu
from jax.experimental.pallas import tpu_sc as plsc

N = 50000
E = 800000
D = 64
H = 32
NS = 16
EPS = E // NS
CHUNK = 80
SUPE = 2000
NSUP = EPS // SUPE
SUPC = SUPE // CHUNK
RING = 4
MAIN_T = SUPC // RING
TAIL_P = MAIN_T * RING
BLKR = 400
NBLK = N // BLKR
LANES = 16


_RB = 10000


def _prep_body(x_ref, W1, b1, W2, b2, mlp_ref):
    x = x_ref[...]
    h = jnp.maximum(jnp.dot(x, W1[...], preferred_element_type=jnp.float32)
                    + b1[...], 0.0)
    m = jnp.maximum(jnp.dot(h, W2[...], preferred_element_type=jnp.float32)
                    + b2[...], 0.0)
    mlp_ref[0] = m[:, :H]
    mlp_ref[1] = m[:, H:]


def _prep_one(x, W1, b1, W2, b2):
    row_spec = pl.BlockSpec((_RB, D), lambda i: (i, 0))
    w_spec = pl.BlockSpec((D, D), lambda i: (0, 0))
    b_spec = pl.BlockSpec((1, D), lambda i: (0, 0))
    cat_spec = pl.BlockSpec((2, _RB, H), lambda i: (0, i, 0))
    return pl.pallas_call(
        _prep_body,
        grid=(N // _RB,),
        in_specs=[row_spec, w_spec, b_spec, w_spec, b_spec],
        out_specs=cat_spec,
        out_shape=jax.ShapeDtypeStruct((2, N, H), jnp.float32),
    )(x, W1, b1.reshape(1, D), W2, b2.reshape(1, D))


def _sc_dir(src_cat, init_feats, gidx, sidx, vals):
    mesh = plsc.VectorSubcoreMesh(core_axis_name="c", subcore_axis_name="s",
                                  num_cores=2, num_subcores=NS)
    out_sd = jax.ShapeDtypeStruct((2 * N, H), jnp.float32)
    cp = pltpu.CompilerParams()
    fields = pltpu.CompilerParams.__dataclass_fields__
    if "needs_layout_passes" in fields:
        cp = dataclasses.replace(cp, needs_layout_passes=False)
    if "use_tc_tiling_on_sc" in fields:
        cp = dataclasses.replace(cp, use_tc_tiling_on_sc=False)

    ring_scratch = []
    for _ in range(RING):
        ring_scratch += [pltpu.VMEM((CHUNK, H), jnp.float32),
                         pltpu.VMEM((CHUNK,), jnp.int32),
                         pltpu.SemaphoreType.DMA,
                         pltpu.SemaphoreType.DMA]

    @functools.partial(
        pl.kernel,
        out_type=out_sd,
        mesh=mesh,
        compiler_params=cp,
        scratch_types=[
            pltpu.VMEM((SUPE,), jnp.int32),
            pltpu.VMEM((SUPE,), jnp.int32),
            pltpu.VMEM((SUPE,), jnp.float32),
            pltpu.VMEM_SHARED((N, H), jnp.float32),
        ] + ring_scratch,
    )
    def kern(src_hbm, init_hbm, gidx_hbm, sidx_hbm, vals_hbm, out_hbm,
             gidxB, sidxB, valsB, accum, *ring_flat):
        c = lax.axis_index("c")
        s = lax.axis_index("s")
        coff = c * N
        ring = [tuple(ring_flat[4 * r:4 * r + 4]) for r in range(RING)]

        def stage(slot, q):
            for k in range(CHUNK // LANES):
                sl = pl.ds(q * CHUNK + k * LANES, LANES)
                dl = pl.ds(k * LANES, LANES)
                slot[1][dl] = sidxB[sl]
                gidxB[sl] = gidxB[sl] + coff

        def issue_gather(q, slot):
            pltpu.async_copy(src_hbm.at[gidxB.at[pl.ds(q * CHUNK, CHUNK)]],
                             slot[0], slot[2])

        def wait_gather(slot):
            pltpu.make_async_copy(src_hbm.at[gidxB.at[pl.ds(0, CHUNK)]],
                                  slot[0], slot[2]).wait()

        def issue_scatter(slot):
            pltpu.async_copy(slot[0], accum.at[slot[1]], slot[3], add=True)

        def wait_scatter(slot):
            pltpu.make_async_copy(slot[0], accum.at[slot[1]], slot[3]).wait()

        def scale(slot, p):
            rows_b = slot[0]
            pbase = p * CHUNK

            @pl.loop(0, CHUNK // LANES)
            def _(i):
                for k in range(LANES):
                    j = i * LANES + k
                    jv = jnp.full((LANES,), pbase + j, jnp.int32)
                    v = plsc.load_gather(valsB, [jv])
                    lo = pl.ds(0, LANES)
                    hi = pl.ds(LANES, LANES)
                    rows_b[j, lo] = rows_b[j, lo] * v
                    rows_b[j, hi] = rows_b[j, hi] * v

        def chunk(p, slot, slot2, prepare):
            wait_gather(slot)
            if prepare:
                q = p + 2

                @pl.when(q < SUPC)
                def _():
                    @pl.when(p >= 2)
                    def _():
                        wait_scatter(slot2)

                    stage(slot2, q)
                    issue_gather(q, slot2)

            scale(slot, p)
            issue_scatter(slot)

        @pl.loop(0, (NBLK + NS - 1) // NS)
        def _(i):
            b = s + i * NS

            @pl.when(b < NBLK)
            def _():
                pltpu.sync_copy(
                    init_hbm.at[pl.ds(b * BLKR, BLKR), pl.ds(c * H, H)],
                    accum.at[pl.ds(b * BLKR, BLKR)])

        plsc.subcore_barrier()
        base_e = s * EPS

        @pl.loop(0, NSUP)
        def _(u):
            sb = base_e + u * SUPE
            pltpu.sync_copy(gidx_hbm.at[pl.ds(sb, SUPE)], gidxB)
            pltpu.sync_copy(sidx_hbm.at[pl.ds(sb, SUPE)], sidxB)
            pltpu.sync_copy(vals_hbm.at[pl.ds(sb, SUPE)], valsB)
            for p0 in range(2):
                stage(ring[p0], p0)
                issue_gather(p0, ring[p0])

            @pl.loop(0, MAIN_T)
            def _(t):
                for r in range(RING):
                    p = t * RING + r
                    chunk(p, ring[r], ring[(r + 2) % RING], prepare=True)

            chunk(TAIL_P, ring[TAIL_P % RING], None, prepare=False)

            for r in range(RING):
                wait_scatter(ring[r])

        plsc.subcore_barrier()

        @pl.loop(0, (NBLK + NS - 1) // NS)
        def _(i):
            b = s + i * NS

            @pl.when(b < NBLK)
            def _():
                pltpu.sync_copy(accum.at[pl.ds(b * BLKR, BLKR)],
                                out_hbm.at[pl.ds(coff + b * BLKR, BLKR)])

        plsc.subcore_barrier()

    return kern(src_cat, init_feats, gidx, sidx, vals)


def _post_body(gc_ref, tilde_ref, W, b, g_ref, beta_ref, ln_ref, j_ref):
    x = jnp.concatenate([gc_ref[0], gc_ref[1]], axis=-1)
    mu = jnp.mean(x, axis=-1, keepdims=True)
    xc = x - mu
    var = jnp.mean(xc * xc, axis=-1, keepdims=True)
    ln = xc * lax.rsqrt(var + 1e-5) * g_ref[...] + beta_ref[...]
    ln_ref[...] = ln
    jin = jnp.concatenate([ln, tilde_ref[...]], axis=-1)
    j_ref[...] = jax.nn.sigmoid(
        jnp.dot(jin, W[...], preferred_element_type=jnp.float32) + b[...])


def _post_one(gc_cat, tilde, jump_W, jump_b, ln_g, ln_b):
    cat_spec = pl.BlockSpec((2, _RB, H), lambda i: (0, i, 0))
    row_spec = pl.BlockSpec((_RB, D), lambda i: (i, 0))
    w_spec = pl.BlockSpec((2 * D, D), lambda i: (0, 0))
    b_spec = pl.BlockSpec((1, D), lambda i: (0, 0))
    out_sd = jax.ShapeDtypeStruct((N, D), jnp.float32)
    return pl.pallas_call(
        _post_body,
        grid=(N // _RB,),
        in_specs=[cat_spec, row_spec, w_spec, b_spec, b_spec, b_spec],
        out_specs=[row_spec, row_spec],
        out_shape=[out_sd, out_sd],
    )(gc_cat, tilde, jump_W, jump_b.reshape(1, D),
      ln_g.reshape(1, D), ln_b.reshape(1, D))


def kernel(cons_features, vars_features, tilde_cons_features,
           tilde_vars_features, edge_indices, edge_attrs,
           cons_W1, cons_b1, cons_W2, cons_b2,
           vars_W1, vars_b1, vars_W2, vars_b2,
           jump_cons_W, jump_cons_b, jump_vars_W, jump_vars_b,
           cons_ln_g, cons_ln_b, vars_ln_g, vars_ln_b):
    rows = edge_indices[0].astype(jnp.int32)
    cols = edge_indices[1].astype(jnp.int32)
    vals = edge_attrs[:, 0]

    mlp_v_cat = _prep_one(vars_features, vars_W1, vars_b1, vars_W2, vars_b2)
    mlp_c_cat = _prep_one(cons_features, cons_W1, cons_b1, cons_W2, cons_b2)

    gc_cons_flat = _sc_dir(mlp_v_cat.reshape(2 * N, H), cons_features,
                           cols, rows, vals)
    gc_vars_flat = _sc_dir(mlp_c_cat.reshape(2 * N, H), vars_features,
                           rows, cols, vals)

    ln_cons, jump_cons = _post_one(gc_cons_flat.reshape(2, N, H),
                                   tilde_cons_features, jump_cons_W,
                                   jump_cons_b, cons_ln_g, cons_ln_b)
    ln_vars, jump_vars = _post_one(gc_vars_flat.reshape(2, N, H),
                                   tilde_vars_features, jump_vars_W,
                                   jump_vars_b, vars_ln_g, vars_ln_b)

    return (ln_cons, ln_vars, jump_cons, jump_vars)


# speedup vs baseline: 1.5874x; 1.0504x over previous
import dataclasses
import functools

import jax
import jax.numpy as jnp
from jax import lax
from jax.experimental import pallas as pl
from jax.experimental.pallas import tpu as pltpu
from jax.experimental.pallas import tpu_sc as plsc

N = 50000
E = 800000
D = 64
H = 32
NS = 16
EPS = E // NS
CHUNK = 80
SUPE = 2000
NSUP = EPS // SUPE
SUPC = SUPE // CHUNK
RING = 4
MAIN_T = SUPC // RING
TAIL_P = MAIN_T * RING
BLKR = 400
NBLK = N // BLKR
LANES = 16


_RB = 10000


def _prep_body(x_ref, W1, b1, W2, b2, mlp_ref):
    x = x_ref[...]
    h = jnp.maximum(jnp.dot(x, W1[...], preferred_element_type=jnp.float32)
                    + b1[...], 0.0)
    m = jnp.maximum(jnp.dot(h, W2[...], preferred_element_type=jnp.float32)
                    + b2[...], 0.0)
    mlp_ref[0] = m[:, :H]
    mlp_ref[1] = m[:, H:]


def _prep_one(x, W1, b1, W2, b2):
    row_spec = pl.BlockSpec((_RB, D), lambda i: (i, 0))
    w_spec = pl.BlockSpec((D, D), lambda i: (0, 0))
    b_spec = pl.BlockSpec((1, D), lambda i: (0, 0))
    cat_spec = pl.BlockSpec((2, _RB, H), lambda i: (0, i, 0))
    return pl.pallas_call(
        _prep_body,
        grid=(N // _RB,),
        in_specs=[row_spec, w_spec, b_spec, w_spec, b_spec],
        out_specs=cat_spec,
        out_shape=jax.ShapeDtypeStruct((2, N, H), jnp.float32),
    )(x, W1, b1.reshape(1, D), W2, b2.reshape(1, D))


def _sc_dir(src_cat, init_feats, gidx, sidx, vals):
    mesh = plsc.VectorSubcoreMesh(core_axis_name="c", subcore_axis_name="s",
                                  num_cores=2, num_subcores=NS)
    out_sd = jax.ShapeDtypeStruct((2 * N, H), jnp.float32)
    cp = pltpu.CompilerParams()
    fields = pltpu.CompilerParams.__dataclass_fields__
    if "needs_layout_passes" in fields:
        cp = dataclasses.replace(cp, needs_layout_passes=False)
    if "use_tc_tiling_on_sc" in fields:
        cp = dataclasses.replace(cp, use_tc_tiling_on_sc=False)

    ring_scratch = []
    for _ in range(RING):
        ring_scratch += [pltpu.VMEM((CHUNK, H), jnp.float32),
                         pltpu.VMEM((CHUNK,), jnp.int32),
                         pltpu.SemaphoreType.DMA,
                         pltpu.SemaphoreType.DMA]

    @functools.partial(
        pl.kernel,
        out_type=out_sd,
        mesh=mesh,
        compiler_params=cp,
        scratch_types=[
            pltpu.VMEM((SUPE,), jnp.int32),
            pltpu.VMEM((SUPE,), jnp.int32),
            pltpu.VMEM((SUPE,), jnp.float32),
            pltpu.VMEM_SHARED((N, H), jnp.float32),
            pltpu.SemaphoreType.DMA,
        ] + ring_scratch,
    )
    def kern(src_hbm, init_hbm, gidx_hbm, sidx_hbm, vals_hbm, out_hbm,
             gidxB, sidxB, valsB, accum, bsem, *ring_flat):
        c = lax.axis_index("c")
        s = lax.axis_index("s")
        coff = c * N
        ring = [tuple(ring_flat[4 * r:4 * r + 4]) for r in range(RING)]

        def stage(slot, q):
            for k in range(CHUNK // LANES):
                sl = pl.ds(q * CHUNK + k * LANES, LANES)
                dl = pl.ds(k * LANES, LANES)
                slot[1][dl] = sidxB[sl]
                gidxB[sl] = gidxB[sl] + coff

        def issue_gather(q, slot):
            pltpu.async_copy(src_hbm.at[gidxB.at[pl.ds(q * CHUNK, CHUNK)]],
                             slot[0], slot[2])

        def wait_gather(slot):
            pltpu.make_async_copy(src_hbm.at[gidxB.at[pl.ds(0, CHUNK)]],
                                  slot[0], slot[2]).wait()

        def issue_scatter(slot):
            pltpu.async_copy(slot[0], accum.at[slot[1]], slot[3], add=True)

        def wait_scatter(slot):
            pltpu.make_async_copy(slot[0], accum.at[slot[1]], slot[3]).wait()

        def scale(slot, p):
            rows_b = slot[0]
            pbase = p * CHUNK

            @pl.loop(0, CHUNK // LANES)
            def _(i):
                for k in range(LANES):
                    j = i * LANES + k
                    jv = jnp.full((LANES,), pbase + j, jnp.int32)
                    v = plsc.load_gather(valsB, [jv])
                    lo = pl.ds(0, LANES)
                    hi = pl.ds(LANES, LANES)
                    rows_b[j, lo] = rows_b[j, lo] * v
                    rows_b[j, hi] = rows_b[j, hi] * v

        def chunk(p, slot, slot2, prepare):
            wait_gather(slot)
            if prepare:
                q = p + 2

                @pl.when(q < SUPC)
                def _():
                    @pl.when(p >= 2)
                    def _():
                        wait_scatter(slot2)

                    stage(slot2, q)
                    issue_gather(q, slot2)

            scale(slot, p)
            issue_scatter(slot)

        @pl.loop(0, (NBLK + NS - 1) // NS)
        def _(i):
            b = s + i * NS

            @pl.when(b < NBLK)
            def _():
                pltpu.async_copy(
                    init_hbm.at[pl.ds(b * BLKR, BLKR), pl.ds(c * H, H)],
                    accum.at[pl.ds(b * BLKR, BLKR)], bsem)

        @pl.loop(0, (NBLK + NS - 1) // NS)
        def _(i):
            b = s + i * NS

            @pl.when(b < NBLK)
            def _():
                pltpu.make_async_copy(
                    init_hbm.at[pl.ds(0, BLKR), pl.ds(c * H, H)],
                    accum.at[pl.ds(0, BLKR)], bsem).wait()

        plsc.subcore_barrier()
        base_e = s * EPS

        @pl.loop(0, NSUP)
        def _(u):
            sb = base_e + u * SUPE
            pltpu.async_copy(gidx_hbm.at[pl.ds(sb, SUPE)], gidxB, bsem)
            pltpu.async_copy(sidx_hbm.at[pl.ds(sb, SUPE)], sidxB, bsem)
            pltpu.async_copy(vals_hbm.at[pl.ds(sb, SUPE)], valsB, bsem)
            pltpu.make_async_copy(gidx_hbm.at[pl.ds(sb, SUPE)], gidxB,
                                  bsem).wait()
            pltpu.make_async_copy(sidx_hbm.at[pl.ds(sb, SUPE)], sidxB,
                                  bsem).wait()
            pltpu.make_async_copy(vals_hbm.at[pl.ds(sb, SUPE)], valsB,
                                  bsem).wait()
            for p0 in range(2):
                stage(ring[p0], p0)
                issue_gather(p0, ring[p0])

            @pl.loop(0, MAIN_T)
            def _(t):
                for r in range(RING):
                    p = t * RING + r
                    chunk(p, ring[r], ring[(r + 2) % RING], prepare=True)

            chunk(TAIL_P, ring[TAIL_P % RING], None, prepare=False)

            for r in range(RING):
                wait_scatter(ring[r])

        plsc.subcore_barrier()

        @pl.loop(0, (NBLK + NS - 1) // NS)
        def _(i):
            b = s + i * NS

            @pl.when(b < NBLK)
            def _():
                pltpu.async_copy(accum.at[pl.ds(b * BLKR, BLKR)],
                                 out_hbm.at[pl.ds(coff + b * BLKR, BLKR)],
                                 bsem)

        @pl.loop(0, (NBLK + NS - 1) // NS)
        def _(i):
            b = s + i * NS

            @pl.when(b < NBLK)
            def _():
                pltpu.make_async_copy(accum.at[pl.ds(0, BLKR)],
                                      out_hbm.at[pl.ds(coff, BLKR)],
                                      bsem).wait()

        plsc.subcore_barrier()

    return kern(src_cat, init_feats, gidx, sidx, vals)


def _post_body(gc_ref, tilde_ref, W, b, g_ref, beta_ref, ln_ref, j_ref):
    x = jnp.concatenate([gc_ref[0], gc_ref[1]], axis=-1)
    mu = jnp.mean(x, axis=-1, keepdims=True)
    xc = x - mu
    var = jnp.mean(xc * xc, axis=-1, keepdims=True)
    ln = xc * lax.rsqrt(var + 1e-5) * g_ref[...] + beta_ref[...]
    ln_ref[...] = ln
    jin = jnp.concatenate([ln, tilde_ref[...]], axis=-1)
    j_ref[...] = jax.nn.sigmoid(
        jnp.dot(jin, W[...], preferred_element_type=jnp.float32) + b[...])


def _post_one(gc_cat, tilde, jump_W, jump_b, ln_g, ln_b):
    cat_spec = pl.BlockSpec((2, _RB, H), lambda i: (0, i, 0))
    row_spec = pl.BlockSpec((_RB, D), lambda i: (i, 0))
    w_spec = pl.BlockSpec((2 * D, D), lambda i: (0, 0))
    b_spec = pl.BlockSpec((1, D), lambda i: (0, 0))
    out_sd = jax.ShapeDtypeStruct((N, D), jnp.float32)
    return pl.pallas_call(
        _post_body,
        grid=(N // _RB,),
        in_specs=[cat_spec, row_spec, w_spec, b_spec, b_spec, b_spec],
        out_specs=[row_spec, row_spec],
        out_shape=[out_sd, out_sd],
    )(gc_cat, tilde, jump_W, jump_b.reshape(1, D),
      ln_g.reshape(1, D), ln_b.reshape(1, D))


def kernel(cons_features, vars_features, tilde_cons_features,
           tilde_vars_features, edge_indices, edge_attrs,
           cons_W1, cons_b1, cons_W2, cons_b2,
           vars_W1, vars_b1, vars_W2, vars_b2,
           jump_cons_W, jump_cons_b, jump_vars_W, jump_vars_b,
           cons_ln_g, cons_ln_b, vars_ln_g, vars_ln_b):
    rows = edge_indices[0].astype(jnp.int32)
    cols = edge_indices[1].astype(jnp.int32)
    vals = edge_attrs[:, 0]

    mlp_v_cat = _prep_one(vars_features, vars_W1, vars_b1, vars_W2, vars_b2)
    mlp_c_cat = _prep_one(cons_features, cons_W1, cons_b1, cons_W2, cons_b2)

    gc_cons_flat = _sc_dir(mlp_v_cat.reshape(2 * N, H), cons_features,
                           cols, rows, vals)
    gc_vars_flat = _sc_dir(mlp_c_cat.reshape(2 * N, H), vars_features,
                           rows, cols, vals)

    ln_cons, jump_cons = _post_one(gc_cons_flat.reshape(2, N, H),
                                   tilde_cons_features, jump_cons_W,
                                   jump_cons_b, cons_ln_g, cons_ln_b)
    ln_vars, jump_vars = _post_one(gc_vars_flat.reshape(2, N, H),
                                   tilde_vars_features, jump_vars_W,
                                   jump_vars_b, vars_ln_g, vars_ln_b)

    return (ln_cons, ln_vars, jump_cons, jump_vars)


# speedup vs baseline: 1.6146x; 1.0171x over previous
import dataclasses
import functools

import jax
import jax.numpy as jnp
from jax import lax
from jax.experimental import pallas as pl
from jax.experimental.pallas import tpu as pltpu
from jax.experimental.pallas import tpu_sc as plsc

N = 50000
E = 800000
D = 64
H = 32
NS = 16
EPS = E // NS
CHUNK = 80
SUPE = 2000
NSUP = EPS // SUPE
SUPC = SUPE // CHUNK
RING = 5
MAIN_T = SUPC // RING
TAIL_P = MAIN_T * RING
BLKR = 400
NBLK = N // BLKR
LANES = 16


_RB = 10000


def _prep_body(x_ref, W1, b1, W2, b2, mlp_ref):
    x = x_ref[...]
    h = jnp.maximum(jnp.dot(x, W1[...], preferred_element_type=jnp.float32)
                    + b1[...], 0.0)
    m = jnp.maximum(jnp.dot(h, W2[...], preferred_element_type=jnp.float32)
                    + b2[...], 0.0)
    mlp_ref[0] = m[:, :H]
    mlp_ref[1] = m[:, H:]


def _prep_one(x, W1, b1, W2, b2):
    row_spec = pl.BlockSpec((_RB, D), lambda i: (i, 0))
    w_spec = pl.BlockSpec((D, D), lambda i: (0, 0))
    b_spec = pl.BlockSpec((1, D), lambda i: (0, 0))
    cat_spec = pl.BlockSpec((2, _RB, H), lambda i: (0, i, 0))
    return pl.pallas_call(
        _prep_body,
        grid=(N // _RB,),
        in_specs=[row_spec, w_spec, b_spec, w_spec, b_spec],
        out_specs=cat_spec,
        out_shape=jax.ShapeDtypeStruct((2, N, H), jnp.float32),
    )(x, W1, b1.reshape(1, D), W2, b2.reshape(1, D))


def _sc_dir(src_cat, init_feats, gidx, sidx, vals):
    mesh = plsc.VectorSubcoreMesh(core_axis_name="c", subcore_axis_name="s",
                                  num_cores=2, num_subcores=NS)
    out_sd = jax.ShapeDtypeStruct((2 * N, H), jnp.float32)
    cp = pltpu.CompilerParams()
    fields = pltpu.CompilerParams.__dataclass_fields__
    if "needs_layout_passes" in fields:
        cp = dataclasses.replace(cp, needs_layout_passes=False)
    if "use_tc_tiling_on_sc" in fields:
        cp = dataclasses.replace(cp, use_tc_tiling_on_sc=False)

    ring_scratch = []
    for _ in range(RING):
        ring_scratch += [pltpu.VMEM((CHUNK, H), jnp.float32),
                         pltpu.VMEM((CHUNK,), jnp.int32),
                         pltpu.SemaphoreType.DMA,
                         pltpu.SemaphoreType.DMA]

    @functools.partial(
        pl.kernel,
        out_type=out_sd,
        mesh=mesh,
        compiler_params=cp,
        scratch_types=[
            pltpu.VMEM((SUPE,), jnp.int32),
            pltpu.VMEM((SUPE,), jnp.int32),
            pltpu.VMEM((SUPE,), jnp.float32),
            pltpu.VMEM_SHARED((N, H), jnp.float32),
            pltpu.SemaphoreType.DMA,
        ] + ring_scratch,
    )
    def kern(src_hbm, init_hbm, gidx_hbm, sidx_hbm, vals_hbm, out_hbm,
             gidxB, sidxB, valsB, accum, bsem, *ring_flat):
        c = lax.axis_index("c")
        s = lax.axis_index("s")
        coff = c * N
        ring = [tuple(ring_flat[4 * r:4 * r + 4]) for r in range(RING)]

        def stage(slot, q):
            for k in range(CHUNK // LANES):
                sl = pl.ds(q * CHUNK + k * LANES, LANES)
                dl = pl.ds(k * LANES, LANES)
                slot[1][dl] = sidxB[sl]
                gidxB[sl] = gidxB[sl] + coff

        def issue_gather(q, slot):
            pltpu.async_copy(src_hbm.at[gidxB.at[pl.ds(q * CHUNK, CHUNK)]],
                             slot[0], slot[2])

        def wait_gather(slot):
            pltpu.make_async_copy(src_hbm.at[gidxB.at[pl.ds(0, CHUNK)]],
                                  slot[0], slot[2]).wait()

        def issue_scatter(slot):
            pltpu.async_copy(slot[0], accum.at[slot[1]], slot[3], add=True)

        def wait_scatter(slot):
            pltpu.make_async_copy(slot[0], accum.at[slot[1]], slot[3]).wait()

        def scale(slot, p):
            rows_b = slot[0]
            pbase = p * CHUNK

            @pl.loop(0, CHUNK // LANES)
            def _(i):
                for k in range(LANES):
                    j = i * LANES + k
                    jv = jnp.full((LANES,), pbase + j, jnp.int32)
                    v = plsc.load_gather(valsB, [jv])
                    lo = pl.ds(0, LANES)
                    hi = pl.ds(LANES, LANES)
                    rows_b[j, lo] = rows_b[j, lo] * v
                    rows_b[j, hi] = rows_b[j, hi] * v

        def chunk(p, slot, slot2, prepare):
            wait_gather(slot)
            if prepare:
                q = p + 3

                @pl.when(q < SUPC)
                def _():
                    @pl.when(p >= 2)
                    def _():
                        wait_scatter(slot2)

                    stage(slot2, q)
                    issue_gather(q, slot2)

            scale(slot, p)
            issue_scatter(slot)

        @pl.loop(0, (NBLK + NS - 1) // NS)
        def _(i):
            b = s + i * NS

            @pl.when(b < NBLK)
            def _():
                pltpu.async_copy(
                    init_hbm.at[pl.ds(b * BLKR, BLKR), pl.ds(c * H, H)],
                    accum.at[pl.ds(b * BLKR, BLKR)], bsem)

        @pl.loop(0, (NBLK + NS - 1) // NS)
        def _(i):
            b = s + i * NS

            @pl.when(b < NBLK)
            def _():
                pltpu.make_async_copy(
                    init_hbm.at[pl.ds(0, BLKR), pl.ds(c * H, H)],
                    accum.at[pl.ds(0, BLKR)], bsem).wait()

        plsc.subcore_barrier()
        base_e = s * EPS

        @pl.loop(0, NSUP)
        def _(u):
            sb = base_e + u * SUPE
            pltpu.async_copy(gidx_hbm.at[pl.ds(sb, SUPE)], gidxB, bsem)
            pltpu.async_copy(sidx_hbm.at[pl.ds(sb, SUPE)], sidxB, bsem)
            pltpu.async_copy(vals_hbm.at[pl.ds(sb, SUPE)], valsB, bsem)
            pltpu.make_async_copy(gidx_hbm.at[pl.ds(sb, SUPE)], gidxB,
                                  bsem).wait()
            pltpu.make_async_copy(sidx_hbm.at[pl.ds(sb, SUPE)], sidxB,
                                  bsem).wait()
            pltpu.make_async_copy(vals_hbm.at[pl.ds(sb, SUPE)], valsB,
                                  bsem).wait()
            for p0 in range(3):
                stage(ring[p0], p0)
                issue_gather(p0, ring[p0])

            @pl.loop(0, MAIN_T)
            def _(t):
                for r in range(RING):
                    p = t * RING + r
                    chunk(p, ring[r], ring[(r + 3) % RING], prepare=True)

            if SUPC % RING:
                chunk(TAIL_P, ring[TAIL_P % RING], None, prepare=False)

            for r in range(RING):
                wait_scatter(ring[r])

        plsc.subcore_barrier()

        @pl.loop(0, (NBLK + NS - 1) // NS)
        def _(i):
            b = s + i * NS

            @pl.when(b < NBLK)
            def _():
                pltpu.async_copy(accum.at[pl.ds(b * BLKR, BLKR)],
                                 out_hbm.at[pl.ds(coff + b * BLKR, BLKR)],
                                 bsem)

        @pl.loop(0, (NBLK + NS - 1) // NS)
        def _(i):
            b = s + i * NS

            @pl.when(b < NBLK)
            def _():
                pltpu.make_async_copy(accum.at[pl.ds(0, BLKR)],
                                      out_hbm.at[pl.ds(coff, BLKR)],
                                      bsem).wait()

        plsc.subcore_barrier()

    return kern(src_cat, init_feats, gidx, sidx, vals)


def _post_body(gc_ref, tilde_ref, W, b, g_ref, beta_ref, ln_ref, j_ref):
    x = jnp.concatenate([gc_ref[0], gc_ref[1]], axis=-1)
    mu = jnp.mean(x, axis=-1, keepdims=True)
    xc = x - mu
    var = jnp.mean(xc * xc, axis=-1, keepdims=True)
    ln = xc * lax.rsqrt(var + 1e-5) * g_ref[...] + beta_ref[...]
    ln_ref[...] = ln
    jin = jnp.concatenate([ln, tilde_ref[...]], axis=-1)
    j_ref[...] = jax.nn.sigmoid(
        jnp.dot(jin, W[...], preferred_element_type=jnp.float32) + b[...])


def _post_one(gc_cat, tilde, jump_W, jump_b, ln_g, ln_b):
    cat_spec = pl.BlockSpec((2, _RB, H), lambda i: (0, i, 0))
    row_spec = pl.BlockSpec((_RB, D), lambda i: (i, 0))
    w_spec = pl.BlockSpec((2 * D, D), lambda i: (0, 0))
    b_spec = pl.BlockSpec((1, D), lambda i: (0, 0))
    out_sd = jax.ShapeDtypeStruct((N, D), jnp.float32)
    return pl.pallas_call(
        _post_body,
        grid=(N // _RB,),
        in_specs=[cat_spec, row_spec, w_spec, b_spec, b_spec, b_spec],
        out_specs=[row_spec, row_spec],
        out_shape=[out_sd, out_sd],
    )(gc_cat, tilde, jump_W, jump_b.reshape(1, D),
      ln_g.reshape(1, D), ln_b.reshape(1, D))


def kernel(cons_features, vars_features, tilde_cons_features,
           tilde_vars_features, edge_indices, edge_attrs,
           cons_W1, cons_b1, cons_W2, cons_b2,
           vars_W1, vars_b1, vars_W2, vars_b2,
           jump_cons_W, jump_cons_b, jump_vars_W, jump_vars_b,
           cons_ln_g, cons_ln_b, vars_ln_g, vars_ln_b):
    rows = edge_indices[0].astype(jnp.int32)
    cols = edge_indices[1].astype(jnp.int32)
    vals = edge_attrs[:, 0]

    mlp_v_cat = _prep_one(vars_features, vars_W1, vars_b1, vars_W2, vars_b2)
    mlp_c_cat = _prep_one(cons_features, cons_W1, cons_b1, cons_W2, cons_b2)

    gc_cons_flat = _sc_dir(mlp_v_cat.reshape(2 * N, H), cons_features,
                           cols, rows, vals)
    gc_vars_flat = _sc_dir(mlp_c_cat.reshape(2 * N, H), vars_features,
                           rows, cols, vals)

    ln_cons, jump_cons = _post_one(gc_cons_flat.reshape(2, N, H),
                                   tilde_cons_features, jump_cons_W,
                                   jump_cons_b, cons_ln_g, cons_ln_b)
    ln_vars, jump_vars = _post_one(gc_vars_flat.reshape(2, N, H),
                                   tilde_vars_features, jump_vars_W,
                                   jump_vars_b, vars_ln_g, vars_ln_b)

    return (ln_cons, ln_vars, jump_cons, jump_vars)
